# baseline (speedup 1.0000x reference)
import jax
import jax.numpy as jnp
from jax.experimental import pallas as pl

NSEL = 300


def _sig_body(x_ref, o_ref):
    o_ref[...] = jax.nn.sigmoid(x_ref[...])


def kernel(obj_logits, obj_boxes, target_sizes):
    B, Q, C = obj_logits.shape
    flat = obj_logits.reshape(50, 819, 128)
    prob = pl.pallas_call(
        _sig_body,
        out_shape=jax.ShapeDtypeStruct(flat.shape, flat.dtype),
        grid=(50,),
        in_specs=[pl.BlockSpec((1, 819, 128), lambda i: (i, 0, 0))],
        out_specs=pl.BlockSpec((1, 819, 128), lambda i: (i, 0, 0)),
    )(flat).reshape(B, Q * C)
    topk_values, topk_indexes = jax.lax.top_k(prob, NSEL)
    scores = topk_values
    topk_boxes = topk_indexes // C
    labels = topk_indexes % C
    cx, cy, w, h = jnp.split(obj_boxes, 4, axis=-1)
    boxes = jnp.concatenate(
        [cx - 0.5 * w, cy - 0.5 * h, cx + 0.5 * w, cy + 0.5 * h], axis=-1)
    boxes = jnp.take_along_axis(boxes, topk_boxes[:, :, None], axis=1)
    img_h = target_sizes[:, 0].astype(boxes.dtype)
    img_w = target_sizes[:, 1].astype(boxes.dtype)
    scale_fct = jnp.stack([img_w, img_h, img_w, img_h], axis=1)
    boxes = boxes * scale_fct[:, None, :]
    return scores, labels, boxes


# speedup vs baseline: 8.9052x; 8.9052x over previous
import jax
import jax.numpy as jnp
from jax import lax
from jax.experimental import pallas as pl
from jax.experimental.pallas import tpu as pltpu
from jax.experimental.pallas import tpu_sc as plsc

NSEL = 300
NCLS = 91
QC = 81900
QC_PAD = 81920
NVEC = QC_PAD // 16
SHIFT = 16
NBUCKET = 16384
NBVEC = NBUCKET // 16
CAND = 512
CVEC = CAND // 16
OUT_PAD = 320
NROW = 64


def _sig_body(x_ref, o_ref):
    o_ref[...] = jax.nn.sigmoid(x_ref[...])


def _sc_topk(prob_hbm, boxes_hbm, scale_hbm, scores_hbm, labels_hbm,
             boxeso_hbm, row_v, hist_v, ck_v, ci_v, boxes_v, scale_v,
             sco_v, lab_v, bxo_v):
    wid = lax.axis_index("s") * 2 + lax.axis_index("c")
    iota = lax.iota(jnp.int32, 16)
    zeros16 = iota * 0
    ones16 = zeros16 + 1

    def do_row(rr, _):
        row = wid * 2 + rr
        pltpu.sync_copy(prob_hbm.at[row], row_v)
        pltpu.sync_copy(boxes_hbm.at[row], boxes_v)
        pltpu.sync_copy(scale_hbm.at[row], scale_v)

        def zero_body(j, c):
            hist_v[pl.ds(j * 16, 16)] = zeros16
            return c

        lax.fori_loop(0, NBVEC, zero_body, 0)

        def hist_body(i, c):
            k = plsc.bitcast(row_v[pl.ds(i * 16, 16)], jnp.int32)
            plsc.addupdate_scatter(hist_v, [k >> SHIFT], ones16)
            return c

        lax.fori_loop(0, NVEC, hist_body, 0)

        def thr_body(j, carry):
            acc, bstar = carry
            jr = (NBVEC - 1) - j
            h = hist_v[pl.ds(jr * 16, 16)]
            s = jnp.sum(h)
            rc = plsc.cumsum(lax.rev(h, (0,)))
            f = jnp.max(plsc.all_reduce_ffs((acc + rc) >= NSEL))
            cand_b = jr * 16 + 15 - f
            cross = (acc < NSEL) & ((acc + s) >= NSEL)
            return (acc + s, jnp.where(cross, cand_b, bstar))

        _, bstar = lax.fori_loop(0, NBVEC, thr_body,
                                 (jnp.int32(0), jnp.int32(0)))

        def init_body(j, c):
            ck_v[pl.ds(j * 16, 16)] = zeros16 - 1
            ci_v[pl.ds(j * 16, 16)] = zeros16
            return c

        lax.fori_loop(0, CVEC, init_body, 0)

        def comp_body(i, off):
            k = plsc.bitcast(row_v[pl.ds(i * 16, 16)], jnp.int32)
            m = (k >> SHIFT) >= bstar
            offc = jnp.minimum(off, CAND - 16)
            plsc.store_compressed(ck_v.at[pl.ds(offc, 16)], k, mask=m)
            plsc.store_compressed(ci_v.at[pl.ds(offc, 16)], iota + i * 16,
                                  mask=m)
            return off + jnp.max(plsc.all_reduce_population_count(m))

        lax.fori_loop(0, NVEC, comp_body, jnp.int32(0))

        for st in range(1, 10):
            kk = 1 << st
            j = kk >> 1
            while j >= 1:
                if j >= 16:
                    jv = j // 16

                    def cross_body(t, c, jv=jv, kk=kk):
                        q = t // jv
                        v = q * (2 * jv) + (t - q * jv)
                        p = v + jv
                        ka = ck_v[pl.ds(v * 16, 16)]
                        ia = ci_v[pl.ds(v * 16, 16)]
                        kb = ck_v[pl.ds(p * 16, 16)]
                        ib = ci_v[pl.ds(p * 16, 16)]
                        a_first = (ka > kb) | ((ka == kb) & (ia < ib))
                        dirf = ((zeros16 + v * 16) & kk) == 0
                        keep = jnp.where(dirf, a_first, ~a_first)
                        ck_v[pl.ds(v * 16, 16)] = jnp.where(keep, ka, kb)
                        ci_v[pl.ds(v * 16, 16)] = jnp.where(keep, ia, ib)
                        ck_v[pl.ds(p * 16, 16)] = jnp.where(keep, kb, ka)
                        ci_v[pl.ds(p * 16, 16)] = jnp.where(keep, ib, ia)
                        return c

                    lax.fori_loop(0, CVEC // 2, cross_body, 0)
                else:
                    perm = iota ^ j
                    lower = (iota & j) == 0

                    def intra_body(v, c, j=j, kk=kk, perm=perm, lower=lower):
                        ks = ck_v[pl.ds(v * 16, 16)]
                        is_ = ci_v[pl.ds(v * 16, 16)]
                        ko = plsc.load_gather(ck_v, [v * 16 + perm])
                        io = plsc.load_gather(ci_v, [v * 16 + perm])
                        s_first = (ks > ko) | ((ks == ko) & (is_ < io))
                        dirf = ((iota + v * 16) & kk) == 0
                        keep = jnp.where(lower == dirf, s_first, ~s_first)
                        ck_v[pl.ds(v * 16, 16)] = jnp.where(keep, ks, ko)
                        ci_v[pl.ds(v * 16, 16)] = jnp.where(keep, is_, io)
                        return c

                    lax.fori_loop(0, CVEC, intra_body, 0)
                j >>= 1

        s0 = scale_v[0]
        s1 = scale_v[1]
        s2 = scale_v[2]
        s3 = scale_v[3]

        def out_body(jj, c):
            kj = ck_v[pl.ds(jj * 16, 16)]
            ij = ci_v[pl.ds(jj * 16, 16)]
            sco_v[pl.ds(jj * 16, 16)] = plsc.bitcast(kj, jnp.float32)
            bq = ij // NCLS
            lab_v[pl.ds(jj * 16, 16)] = ij - bq * NCLS
            bqc = jnp.minimum(bq, QC // NCLS - 1)
            cx = plsc.load_gather(boxes_v, [zeros16, bqc])
            cy = plsc.load_gather(boxes_v, [ones16, bqc])
            w = plsc.load_gather(boxes_v, [zeros16 + 2, bqc])
            h = plsc.load_gather(boxes_v, [zeros16 + 3, bqc])
            bxo_v[0, pl.ds(jj * 16, 16)] = (cx - 0.5 * w) * s0
            bxo_v[1, pl.ds(jj * 16, 16)] = (cy - 0.5 * h) * s1
            bxo_v[2, pl.ds(jj * 16, 16)] = (cx + 0.5 * w) * s2
            bxo_v[3, pl.ds(jj * 16, 16)] = (cy + 0.5 * h) * s3
            return c

        lax.fori_loop(0, OUT_PAD // 16, out_body, 0)
        pltpu.sync_copy(sco_v, scores_hbm.at[row])
        pltpu.sync_copy(lab_v, labels_hbm.at[row])
        pltpu.sync_copy(bxo_v, boxeso_hbm.at[row])
        return 0

    lax.fori_loop(0, 2, do_row, 0)


def kernel(obj_logits, obj_boxes, target_sizes):
    B, Q, C = obj_logits.shape
    flat = obj_logits.reshape(B, Q * C)
    flat = jnp.pad(flat, ((0, 0), (0, QC_PAD - Q * C)),
                   constant_values=-1e30)
    prob = pl.pallas_call(
        _sig_body,
        out_shape=jax.ShapeDtypeStruct((B * QC_PAD // 128, 128), jnp.float32),
        grid=(10,),
        in_specs=[pl.BlockSpec((B * QC_PAD // 1280, 128), lambda i: (i, 0))],
        out_specs=pl.BlockSpec((B * QC_PAD // 1280, 128), lambda i: (i, 0)),
    )(flat.reshape(B * QC_PAD // 128, 128)).reshape(B, QC_PAD)

    boxes_t = obj_boxes.transpose(0, 2, 1)
    img_h = target_sizes[:, 0].astype(jnp.float32)
    img_w = target_sizes[:, 1].astype(jnp.float32)
    scale = jnp.stack([img_w, img_h, img_w, img_h], axis=1)
    scale16 = jnp.broadcast_to(scale[:, :, None], (B, 4, 16))

    sc = pl.kernel(
        _sc_topk,
        out_type=[
            jax.ShapeDtypeStruct((NROW, OUT_PAD), jnp.float32),
            jax.ShapeDtypeStruct((NROW, OUT_PAD), jnp.int32),
            jax.ShapeDtypeStruct((NROW, 4, OUT_PAD), jnp.float32),
        ],
        mesh=plsc.VectorSubcoreMesh(core_axis_name="c", subcore_axis_name="s"),
        compiler_params=pltpu.CompilerParams(needs_layout_passes=False),
        scratch_types=[
            pltpu.VMEM((QC_PAD,), jnp.float32),
            pltpu.VMEM((NBUCKET,), jnp.int32),
            pltpu.VMEM((CAND,), jnp.int32),
            pltpu.VMEM((CAND,), jnp.int32),
            pltpu.VMEM((4, Q), jnp.float32),
            pltpu.VMEM((4, 16), jnp.float32),
            pltpu.VMEM((OUT_PAD,), jnp.float32),
            pltpu.VMEM((OUT_PAD,), jnp.int32),
            pltpu.VMEM((4, OUT_PAD), jnp.float32),
        ],
    )
    scores_p, labels_p, boxes_p = sc(prob, boxes_t, scale16)
    return (scores_p[:, :NSEL], labels_p[:, :NSEL],
            boxes_p.transpose(0, 2, 1)[:, :NSEL, :])


# speedup vs baseline: 10.5586x; 1.1857x over previous
import jax
import jax.numpy as jnp
from jax import lax
from jax.experimental import pallas as pl
from jax.experimental.pallas import tpu as pltpu
from jax.experimental.pallas import tpu_sc as plsc

NSEL = 300
NCLS = 91
QC = 81900
QC_PAD = 81920
NVEC = QC_PAD // 16
SHIFT = 16
NBUCKET = 16384
NBVEC = NBUCKET // 16
CAND = 512
CVEC = CAND // 16
OUT_PAD = 320
NROW = 64


def _sig_body(x_ref, o_ref):
    o_ref[:, :QC] = jax.nn.sigmoid(x_ref[...])
    o_ref[:, QC:] = jnp.zeros((8, QC_PAD - QC), jnp.float32)


def _sc_topk(prob_hbm, boxes_hbm, scale_hbm, scores_hbm, labels_hbm,
             boxeso_hbm, row_v, hist_v, ck_v, ci_v, boxes_v, scale_v,
             sco_v, lab_v, bxo_v):
    wid = lax.axis_index("s") * 2 + lax.axis_index("c")
    iota = lax.iota(jnp.int32, 16)
    zeros16 = iota * 0
    ones16 = zeros16 + 1

    def do_row(rr, _):
        row = wid * 2 + rr
        pltpu.sync_copy(prob_hbm.at[row], row_v)
        pltpu.sync_copy(boxes_hbm.at[row], boxes_v)
        pltpu.sync_copy(scale_hbm.at[row], scale_v)

        def _zero(j, c):
            hist_v[pl.ds(j * 16, 16)] = zeros16
            return c
        lax.fori_loop(0, NBVEC, _zero, 0)

        def _hist(i, c):
            k = plsc.bitcast(row_v[pl.ds(i * 16, 16)], jnp.int32)
            plsc.addupdate_scatter(hist_v, [k >> SHIFT], ones16)
            return c
        lax.fori_loop(0, NVEC, _hist, 0)

        def thr_cond(carry):
            jr, acc, _ = carry
            return (acc < NSEL) & (jr >= 0)

        def thr_body(carry):
            jr, acc, bstar = carry
            h = hist_v[pl.ds(jr * 16, 16)]
            s = jnp.sum(h)
            rc = plsc.cumsum(lax.rev(h, (0,)))
            f = jnp.max(plsc.all_reduce_ffs((acc + rc) >= NSEL))
            cand_b = jr * 16 + 15 - f
            cross = (acc + s) >= NSEL
            return (jr - 1, acc + s, jnp.where(cross, cand_b, bstar))

        _, _, bstar = lax.while_loop(
            thr_cond, thr_body,
            (jnp.int32(NBVEC - 1), jnp.int32(0), jnp.int32(0)))

        def _init(j, c):
            ck_v[pl.ds(j * 16, 16)] = zeros16 - 1
            ci_v[pl.ds(j * 16, 16)] = zeros16
            return c
        lax.fori_loop(0, CVEC, _init, 0)

        def _compact(i, off):
            k = plsc.bitcast(row_v[pl.ds(i * 16, 16)], jnp.int32)
            m = (k >> SHIFT) >= bstar
            offc = jnp.minimum(off, CAND - 16)
            plsc.store_compressed(ck_v.at[pl.ds(offc, 16)], k, mask=m)
            plsc.store_compressed(ci_v.at[pl.ds(offc, 16)], iota + i * 16,
                                  mask=m)
            return off + jnp.max(plsc.all_reduce_population_count(m))
        lax.fori_loop(0, NVEC, _compact, jnp.int32(0))

        for st in range(1, 10):
            kk = 1 << st
            j = kk >> 1
            while j >= 1:
                if j >= 16:
                    jv = j // 16

                    def _cross(t, c, jv=jv, kk=kk):
                        q = t // jv
                        v = q * (2 * jv) + (t - q * jv)
                        p = v + jv
                        ka = ck_v[pl.ds(v * 16, 16)]
                        ia = ci_v[pl.ds(v * 16, 16)]
                        kb = ck_v[pl.ds(p * 16, 16)]
                        ib = ci_v[pl.ds(p * 16, 16)]
                        a_first = (ka > kb) | ((ka == kb) & (ia < ib))
                        dirf = ((zeros16 + v * 16) & kk) == 0
                        keep = jnp.where(dirf, a_first, ~a_first)
                        ck_v[pl.ds(v * 16, 16)] = jnp.where(keep, ka, kb)
                        ci_v[pl.ds(v * 16, 16)] = jnp.where(keep, ia, ib)
                        ck_v[pl.ds(p * 16, 16)] = jnp.where(keep, kb, ka)
                        ci_v[pl.ds(p * 16, 16)] = jnp.where(keep, ib, ia)
                        return c
                    lax.fori_loop(0, CVEC // 2, _cross, 0)
                else:
                    perm = iota ^ j
                    lower = (iota & j) == 0

                    def _intra(v, c, j=j, kk=kk, perm=perm, lower=lower):
                        ks = ck_v[pl.ds(v * 16, 16)]
                        is_ = ci_v[pl.ds(v * 16, 16)]
                        ko = plsc.load_gather(ck_v, [v * 16 + perm])
                        io = plsc.load_gather(ci_v, [v * 16 + perm])
                        s_first = (ks > ko) | ((ks == ko) & (is_ < io))
                        dirf = ((iota + v * 16) & kk) == 0
                        keep = jnp.where(lower == dirf, s_first, ~s_first)
                        ck_v[pl.ds(v * 16, 16)] = jnp.where(keep, ks, ko)
                        ci_v[pl.ds(v * 16, 16)] = jnp.where(keep, is_, io)
                        return c
                    lax.fori_loop(0, CVEC, _intra, 0)
                j >>= 1

        s0 = scale_v[0]
        s1 = scale_v[1]
        s2 = scale_v[2]
        s3 = scale_v[3]

        def _emit(jj, c):
            kj = ck_v[pl.ds(jj * 16, 16)]
            ij = ci_v[pl.ds(jj * 16, 16)]
            sco_v[pl.ds(jj * 16, 16)] = plsc.bitcast(kj, jnp.float32)
            bq = ij // NCLS
            lab_v[pl.ds(jj * 16, 16)] = ij - bq * NCLS
            bqc = jnp.minimum(bq, QC // NCLS - 1)
            b4 = bqc * 4
            cx = plsc.load_gather(boxes_v, [b4])
            cy = plsc.load_gather(boxes_v, [b4 + 1])
            w = plsc.load_gather(boxes_v, [b4 + 2])
            h = plsc.load_gather(boxes_v, [b4 + 3])
            bxo_v[0, pl.ds(jj * 16, 16)] = (cx - 0.5 * w) * s0
            bxo_v[1, pl.ds(jj * 16, 16)] = (cy - 0.5 * h) * s1
            bxo_v[2, pl.ds(jj * 16, 16)] = (cx + 0.5 * w) * s2
            bxo_v[3, pl.ds(jj * 16, 16)] = (cy + 0.5 * h) * s3
            return c
        lax.fori_loop(0, OUT_PAD // 16, _emit, 0)

        pltpu.sync_copy(sco_v, scores_hbm.at[row])
        pltpu.sync_copy(lab_v, labels_hbm.at[row])
        pltpu.sync_copy(bxo_v, boxeso_hbm.at[row])
        return 0

    lax.fori_loop(0, 2, do_row, 0)


def kernel(obj_logits, obj_boxes, target_sizes):
    B, Q, C = obj_logits.shape
    flat = obj_logits.reshape(B, Q * C)
    prob = pl.pallas_call(
        _sig_body,
        out_shape=jax.ShapeDtypeStruct((B, QC_PAD), jnp.float32),
        grid=(B // 8,),
        in_specs=[pl.BlockSpec((8, QC), lambda i: (i, 0))],
        out_specs=pl.BlockSpec((8, QC_PAD), lambda i: (i, 0)),
    )(flat)

    img_h = target_sizes[:, 0].astype(jnp.float32)
    img_w = target_sizes[:, 1].astype(jnp.float32)
    scale = jnp.stack([img_w, img_h, img_w, img_h], axis=1)
    scale16 = jnp.broadcast_to(scale[:, :, None], (B, 4, 16))

    sc = pl.kernel(
        _sc_topk,
        out_type=[
            jax.ShapeDtypeStruct((NROW, OUT_PAD), jnp.float32),
            jax.ShapeDtypeStruct((NROW, OUT_PAD), jnp.int32),
            jax.ShapeDtypeStruct((NROW, 4, OUT_PAD), jnp.float32),
        ],
        mesh=plsc.VectorSubcoreMesh(core_axis_name="c", subcore_axis_name="s"),
        compiler_params=pltpu.CompilerParams(needs_layout_passes=False),
        scratch_types=[
            pltpu.VMEM((QC_PAD,), jnp.float32),
            pltpu.VMEM((NBUCKET,), jnp.int32),
            pltpu.VMEM((CAND,), jnp.int32),
            pltpu.VMEM((CAND,), jnp.int32),
            pltpu.VMEM((4 * Q,), jnp.float32),
            pltpu.VMEM((4, 16), jnp.float32),
            pltpu.VMEM((OUT_PAD,), jnp.float32),
            pltpu.VMEM((OUT_PAD,), jnp.int32),
            pltpu.VMEM((4, OUT_PAD), jnp.float32),
        ],
    )
    scores_p, labels_p, boxes_p = sc(prob, obj_boxes.reshape(B, 4 * Q), scale16)
    return (scores_p[:, :NSEL], labels_p[:, :NSEL],
            boxes_p.transpose(0, 2, 1)[:, :NSEL, :])


# speedup vs baseline: 18.1818x; 1.7220x over previous
import jax
import jax.numpy as jnp
from jax import lax
from jax.experimental import pallas as pl
from jax.experimental.pallas import tpu as pltpu
from jax.experimental.pallas import tpu_sc as plsc

NSEL = 300
NCLS = 91
QC = 81900
QC_PAD = 81920
NVEC = QC_PAD // 16
SHIFT = 16
NBUCKET = 16384
NBVEC = NBUCKET // 16
CAND = 512
CVEC = CAND // 16
OUT_PAD = 320
NROW = 64


def _sig_body(x_ref, o_ref):
    o_ref[:, :QC] = jax.nn.sigmoid(x_ref[...])
    o_ref[:, QC:] = jnp.zeros((8, QC_PAD - QC), jnp.float32)


def _sc_topk(prob_hbm, boxes_hbm, scale_hbm, scores_hbm, labels_hbm,
             boxeso_hbm, row_v, hist_v, ck_v, ci_v, boxes_v, scale_v,
             sco_v, lab_v, bxo_v):
    wid = lax.axis_index("s") * 2 + lax.axis_index("c")
    iota = lax.iota(jnp.int32, 16)
    zeros16 = iota * 0
    ones16 = zeros16 + 1

    def do_row(rr, _):
        row = wid * 2 + rr
        pltpu.sync_copy(prob_hbm.at[row], row_v)
        pltpu.sync_copy(boxes_hbm.at[row], boxes_v)
        pltpu.sync_copy(scale_hbm.at[row], scale_v)

        @plsc.parallel_loop(0, NBVEC, unroll=8)
        def _zero(j):
            hist_v[pl.ds(j * 16, 16)] = zeros16

        @plsc.parallel_loop(0, NVEC, unroll=8)
        def _hist(i):
            k = plsc.bitcast(row_v[pl.ds(i * 16, 16)], jnp.int32)
            plsc.addupdate_scatter(hist_v, [k >> SHIFT], ones16)

        def thr_cond(carry):
            jr, acc, _ = carry
            return (acc < NSEL) & (jr >= 0)

        def thr_body(carry):
            jr, acc, bstar = carry
            h = hist_v[pl.ds(jr * 16, 16)]
            s = jnp.sum(h)
            rc = plsc.cumsum(lax.rev(h, (0,)))
            f = jnp.max(plsc.all_reduce_ffs((acc + rc) >= NSEL))
            cand_b = jr * 16 + 15 - f
            cross = (acc + s) >= NSEL
            return (jr - 1, acc + s, jnp.where(cross, cand_b, bstar))

        _, _, bstar = lax.while_loop(
            thr_cond, thr_body,
            (jnp.int32(NBVEC - 1), jnp.int32(0), jnp.int32(0)))

        @plsc.parallel_loop(0, CVEC, unroll=4)
        def _init(j):
            ck_v[pl.ds(j * 16, 16)] = zeros16 - 1
            ci_v[pl.ds(j * 16, 16)] = zeros16

        @plsc.parallel_loop(0, NVEC, unroll=4, carry=jnp.int32(0))
        def _compact(i, off):
            k = plsc.bitcast(row_v[pl.ds(i * 16, 16)], jnp.int32)
            m = (k >> SHIFT) >= bstar
            offc = jnp.minimum(off, CAND - 16)
            plsc.store_compressed(ck_v.at[pl.ds(offc, 16)], k, mask=m)
            plsc.store_compressed(ci_v.at[pl.ds(offc, 16)], iota + i * 16,
                                  mask=m)
            return off + jnp.max(plsc.all_reduce_population_count(m))

        for st in range(1, 10):
            kk = 1 << st
            j = kk >> 1
            while j >= 1:
                if j >= 16:
                    jv = j // 16

                    def _cross(t, c, jv=jv, kk=kk):
                        q = t // jv
                        v = q * (2 * jv) + (t - q * jv)
                        p = v + jv
                        ka = ck_v[pl.ds(v * 16, 16)]
                        ia = ci_v[pl.ds(v * 16, 16)]
                        kb = ck_v[pl.ds(p * 16, 16)]
                        ib = ci_v[pl.ds(p * 16, 16)]
                        a_first = (ka > kb) | ((ka == kb) & (ia < ib))
                        dirf = ((zeros16 + v * 16) & kk) == 0
                        keep = jnp.where(dirf, a_first, ~a_first)
                        ck_v[pl.ds(v * 16, 16)] = jnp.where(keep, ka, kb)
                        ci_v[pl.ds(v * 16, 16)] = jnp.where(keep, ia, ib)
                        ck_v[pl.ds(p * 16, 16)] = jnp.where(keep, kb, ka)
                        ci_v[pl.ds(p * 16, 16)] = jnp.where(keep, ib, ia)
                        return c
                    lax.fori_loop(0, CVEC // 2, _cross, 0)
                else:
                    perm = iota ^ j
                    lower = (iota & j) == 0

                    def _intra(v, c, j=j, kk=kk, perm=perm, lower=lower):
                        ks = ck_v[pl.ds(v * 16, 16)]
                        is_ = ci_v[pl.ds(v * 16, 16)]
                        ko = plsc.load_gather(ck_v, [v * 16 + perm])
                        io = plsc.load_gather(ci_v, [v * 16 + perm])
                        s_first = (ks > ko) | ((ks == ko) & (is_ < io))
                        dirf = ((iota + v * 16) & kk) == 0
                        keep = jnp.where(lower == dirf, s_first, ~s_first)
                        ck_v[pl.ds(v * 16, 16)] = jnp.where(keep, ks, ko)
                        ci_v[pl.ds(v * 16, 16)] = jnp.where(keep, is_, io)
                        return c
                    lax.fori_loop(0, CVEC, _intra, 0)
                j >>= 1

        s0 = scale_v[0]
        s1 = scale_v[1]
        s2 = scale_v[2]
        s3 = scale_v[3]

        def _emit(jj, c):
            kj = ck_v[pl.ds(jj * 16, 16)]
            ij = ci_v[pl.ds(jj * 16, 16)]
            sco_v[pl.ds(jj * 16, 16)] = plsc.bitcast(kj, jnp.float32)
            bq = ij // NCLS
            lab_v[pl.ds(jj * 16, 16)] = ij - bq * NCLS
            bqc = jnp.minimum(bq, QC // NCLS - 1)
            b4 = bqc * 4
            cx = plsc.load_gather(boxes_v, [b4])
            cy = plsc.load_gather(boxes_v, [b4 + 1])
            w = plsc.load_gather(boxes_v, [b4 + 2])
            h = plsc.load_gather(boxes_v, [b4 + 3])
            bxo_v[0, pl.ds(jj * 16, 16)] = (cx - 0.5 * w) * s0
            bxo_v[1, pl.ds(jj * 16, 16)] = (cy - 0.5 * h) * s1
            bxo_v[2, pl.ds(jj * 16, 16)] = (cx + 0.5 * w) * s2
            bxo_v[3, pl.ds(jj * 16, 16)] = (cy + 0.5 * h) * s3
            return c
        lax.fori_loop(0, OUT_PAD // 16, _emit, 0)

        pltpu.sync_copy(sco_v, scores_hbm.at[row])
        pltpu.sync_copy(lab_v, labels_hbm.at[row])
        pltpu.sync_copy(bxo_v, boxeso_hbm.at[row])
        return 0

    lax.fori_loop(0, 2, do_row, 0)


def kernel(obj_logits, obj_boxes, target_sizes):
    B, Q, C = obj_logits.shape
    flat = obj_logits.reshape(B, Q * C)
    prob = pl.pallas_call(
        _sig_body,
        out_shape=jax.ShapeDtypeStruct((B, QC_PAD), jnp.float32),
        grid=(B // 8,),
        in_specs=[pl.BlockSpec((8, QC), lambda i: (i, 0))],
        out_specs=pl.BlockSpec((8, QC_PAD), lambda i: (i, 0)),
    )(flat)

    img_h = target_sizes[:, 0].astype(jnp.float32)
    img_w = target_sizes[:, 1].astype(jnp.float32)
    scale = jnp.stack([img_w, img_h, img_w, img_h], axis=1)
    scale16 = jnp.broadcast_to(scale[:, :, None], (B, 4, 16))

    sc = pl.kernel(
        _sc_topk,
        out_type=[
            jax.ShapeDtypeStruct((NROW, OUT_PAD), jnp.float32),
            jax.ShapeDtypeStruct((NROW, OUT_PAD), jnp.int32),
            jax.ShapeDtypeStruct((NROW, 4, OUT_PAD), jnp.float32),
        ],
        mesh=plsc.VectorSubcoreMesh(core_axis_name="c", subcore_axis_name="s"),
        compiler_params=pltpu.CompilerParams(needs_layout_passes=False),
        scratch_types=[
            pltpu.VMEM((QC_PAD,), jnp.float32),
            pltpu.VMEM((NBUCKET,), jnp.int32),
            pltpu.VMEM((CAND,), jnp.int32),
            pltpu.VMEM((CAND,), jnp.int32),
            pltpu.VMEM((4 * Q,), jnp.float32),
            pltpu.VMEM((4, 16), jnp.float32),
            pltpu.VMEM((OUT_PAD,), jnp.float32),
            pltpu.VMEM((OUT_PAD,), jnp.int32),
            pltpu.VMEM((4, OUT_PAD), jnp.float32),
        ],
    )
    scores_p, labels_p, boxes_p = sc(prob, obj_boxes.reshape(B, 4 * Q), scale16)
    return (scores_p[:, :NSEL], labels_p[:, :NSEL],
            boxes_p.transpose(0, 2, 1)[:, :NSEL, :])
